# baseline (speedup 1.0000x reference)
import numpy as np
import jax
import jax.numpy as jnp
from jax.experimental import pallas as pl
from jax.experimental.pallas import tpu as pltpu

N_KPS = 8
N_NODES = 24
ER = 64
NR = 64
EE = 32
ATTN = 32


def _edge_structure():
    src, dst = [], []
    for i in range(N_NODES):
        for j in range(N_NODES):
            if i == j:
                continue
            src.append(i)
            dst.append(j)
    src = np.array(src)
    dst = np.array(dst)
    e = len(src)
    scat = np.zeros((e, N_NODES), np.float32)
    scat[np.arange(e), src] = 1.0
    intra = (src // N_KPS == dst // N_KPS).astype(np.float32)[:, None]
    return src, dst, scat, intra


_SRC, _DST, _SCAT, _M_INTRA = _edge_structure()
N_SPATIAL = len(_SRC)


def _srnn_kernel(nodes_ref, et_ref, es_ref, scat_ref, scat_t_ref, mi_ref,
                 me_ref, seconst_ref, w_te_ref, b_te_ref, te_wih_ref,
                 te_whh_ref, te_b_ref, w_se_d_ref, w_se_l_ref, se_wih_ref,
                 se_whh_ref, se_b_ref, wq_ref, wki_ref, wke_ref, bqi_ref,
                 bqe_ref, wsi_ref, wse_ref, w_ne_ref, b_ne_ref, w_ee_t_ref,
                 w_ee_i_ref, w_ee_e_ref, b_ee_ref, nd_wih_n_ref, nd_wih_e_ref,
                 nd_whh_ref, nd_b_ref, w_out_ref, b_out_ref, out_ref):
    T = nodes_ref.shape[1]
    E = es_ref.shape[2]
    N = nodes_ref.shape[2]

    scat = scat_ref[...]
    scat_t = scat_t_ref[...]
    m_i = mi_ref[...]
    m_e = me_ref[...]
    se_const = seconst_ref[...]

    def lstm(pre, h, c, whh_ref):
        g = pre + h @ whh_ref[...]
        i = jax.nn.sigmoid(g[:, 0 * ER:1 * ER])
        f = jax.nn.sigmoid(g[:, 1 * ER:2 * ER])
        gg = jnp.tanh(g[:, 2 * ER:3 * ER])
        o = jax.nn.sigmoid(g[:, 3 * ER:4 * ER])
        c2 = f * c + i * gg
        h2 = o * jnp.tanh(c2)
        return h2, c2

    def attend(q_e, h_spat, wk_ref, bqk_ref, ws_ref, mask):
        k = h_spat @ wk_ref[...]
        s = jnp.tanh(q_e + k + bqk_ref[...]) @ ws_ref[...]
        s = s - jnp.max(s)
        ex = jnp.exp(s) * mask
        den = scat_t @ ex
        w = ex / (scat @ den)
        return scat_t @ (w * h_spat)

    def step(t, carry):
        h_temp, c_temp, h_spat, c_spat, h_node, c_node = carry

        et = et_ref[0, t]
        te_in = jax.nn.relu(et @ w_te_ref[...] + b_te_ref[...])
        h_temp, c_temp = lstm(te_in @ te_wih_ref[...] + te_b_ref[...],
                              h_temp, c_temp, te_whh_ref)

        disp = es_ref[0, t]
        dist = jnp.sqrt(jnp.sum(disp * disp, axis=1, keepdims=True))
        dist = jnp.maximum(dist, 1e-6)
        se_pre = ((disp / dist) @ w_se_d_ref[...]
                  + jnp.log(dist) * w_se_l_ref[...] + se_const)
        se_in = jax.nn.relu(se_pre)
        h_spat, c_spat = lstm(se_in @ se_wih_ref[...] + se_b_ref[...],
                              h_spat, c_spat, se_whh_ref)

        q = h_temp @ wq_ref[...]
        q_e = scat @ q
        h_ia = attend(q_e, h_spat, wki_ref, bqi_ref, wsi_ref, m_i)
        h_ea = attend(q_e, h_spat, wke_ref, bqe_ref, wse_ref, m_e)

        node_in = jax.nn.relu(nodes_ref[0, t] @ w_ne_ref[...] + b_ne_ref[...])
        edge_in = jax.nn.relu(h_temp @ w_ee_t_ref[...] + h_ia @ w_ee_i_ref[...]
                              + h_ea @ w_ee_e_ref[...] + b_ee_ref[...])
        pre_n = (node_in @ nd_wih_n_ref[...] + edge_in @ nd_wih_e_ref[...]
                 + nd_b_ref[...])
        h_node, c_node = lstm(pre_n, h_node, c_node, nd_whh_ref)

        out_ref[0, t] = h_node @ w_out_ref[...] + b_out_ref[...]
        return h_temp, c_temp, h_spat, c_spat, h_node, c_node

    z = jnp.zeros((N, ER), jnp.float32)
    ze = jnp.zeros((E, ER), jnp.float32)
    zn = jnp.zeros((N, NR), jnp.float32)
    jax.lax.fori_loop(0, T, step, (z, z, ze, ze, zn, zn))


def kernel(nodes, edges_temporal, edges_spatial, params):
    p = params
    B, T, N, _ = nodes.shape
    E = edges_spatial.shape[2]

    scat = jnp.asarray(_SCAT)
    scat_t = jnp.asarray(_SCAT.T.copy())
    m_i = jnp.asarray(_M_INTRA)
    m_e = 1.0 - m_i

    kp = p['kp_emb']
    w_se = p['W_se']
    se_const = (kp[_SRC % N_KPS] @ w_se[3:3 + N_KPS]
                + kp[_DST % N_KPS] @ w_se[3 + N_KPS:3 + 2 * N_KPS]
                + p['b_se'][None, :])

    def r2(x):
        return x.reshape(1, -1)

    weights = (
        scat, scat_t, m_i, m_e, se_const,
        p['W_te'], r2(p['b_te']),
        p['te_Wih'], p['te_Whh'], r2(p['te_bih'] + p['te_bhh']),
        w_se[0:2], w_se[2:3],
        p['se_Wih'], p['se_Whh'], r2(p['se_bih'] + p['se_bhh']),
        p['Wq'], p['Wki'], p['Wke'],
        r2(p['bq'] + p['bki']), r2(p['bq'] + p['bke']),
        p['Ws_intra'], p['Ws_inter'],
        p['W_ne'], r2(p['b_ne']),
        p['W_ee'][0:ER], p['W_ee'][ER:2 * ER], p['W_ee'][2 * ER:3 * ER],
        r2(p['b_ee']),
        p['nd_Wih'][0:EE], p['nd_Wih'][EE:2 * EE],
        p['nd_Whh'], r2(p['nd_bih'] + p['nd_bhh']),
        p['W_out'], r2(p['b_out']),
    )

    def full(x):
        return pl.BlockSpec(x.shape, lambda b: (0,) * x.ndim)

    in_specs = [
        pl.BlockSpec((1, T, N, 2), lambda b: (b, 0, 0, 0)),
        pl.BlockSpec((1, T, N, 2), lambda b: (b, 0, 0, 0)),
        pl.BlockSpec((1, T, E, 2), lambda b: (b, 0, 0, 0)),
    ] + [full(w) for w in weights]

    out = pl.pallas_call(
        _srnn_kernel,
        grid=(B,),
        in_specs=in_specs,
        out_specs=pl.BlockSpec((1, T, N, 5), lambda b: (b, 0, 0, 0)),
        out_shape=jax.ShapeDtypeStruct((B, T, N, 5), jnp.float32),
        compiler_params=pltpu.CompilerParams(
            dimension_semantics=("arbitrary",)),
    )(nodes, edges_temporal, edges_spatial, *weights)
    return out
